# Initial kernel scaffold; baseline (speedup 1.0000x reference)
import jax
import jax.numpy as jnp
from jax.experimental import pallas as pl


def kernel(idx, targets, token_embedding_table):
    raise NotImplementedError("write your pallas kernel here")


# speedup vs baseline: 1.0149x; 1.0149x over previous
import functools

import jax
import jax.numpy as jnp
from jax import lax
from jax.experimental import pallas as pl
from jax.experimental.pallas import tpu as pltpu
from jax.experimental.pallas import tpu_sc as plsc

_VOCAB = 1000
_B = 1024
_L = 50
_TOTAL = _B * _L
_NW = 32
_PER_W = _TOTAL // _NW
_CHUNK = 64
_NCHUNK = _PER_W // _CHUNK

_mesh = plsc.VectorSubcoreMesh(core_axis_name="c", subcore_axis_name="s")


@functools.partial(
    pl.kernel,
    mesh=_mesh,
    out_type=jax.ShapeDtypeStruct((_TOTAL, _VOCAB), jnp.float32),
    scratch_types=[
        pltpu.VMEM((_NCHUNK, _CHUNK), jnp.int32),
        pltpu.VMEM((_CHUNK, _VOCAB), jnp.float32),
        pltpu.SemaphoreType.DMA,
    ],
    compiler_params=pltpu.CompilerParams(use_tc_tiling_on_sc=False),
)
def _embed(idx_hbm, table_hbm, out_hbm, idx_v, rows_v, gsem):
    wid = lax.axis_index("s") * 2 + lax.axis_index("c")
    base = wid * _PER_W
    pltpu.sync_copy(idx_hbm.at[wid], idx_v)

    def body(g, carry):
        pltpu.async_copy(table_hbm.at[idx_v.at[g]], rows_v, gsem).wait()
        pltpu.sync_copy(rows_v, out_hbm.at[pl.ds(base + g * _CHUNK, _CHUNK)])
        return carry

    lax.fori_loop(0, _NCHUNK, body, 0)


def kernel(idx, targets, token_embedding_table):
    del targets
    idx3 = idx.reshape(_NW, _NCHUNK, _CHUNK).astype(jnp.int32)
    out = _embed(idx3, token_embedding_table)
    return out.reshape(_B, _L, _VOCAB)


# speedup vs baseline: 1.0286x; 1.0135x over previous
import functools

import jax
import jax.numpy as jnp
from jax import lax
from jax.experimental import pallas as pl
from jax.experimental.pallas import tpu as pltpu
from jax.experimental.pallas import tpu_sc as plsc

_VOCAB = 1000
_B = 1024
_L = 50
_TOTAL = _B * _L
_NW = 32
_PER_W = _TOTAL // _NW
_CHUNK = 40
_NCHUNK = _PER_W // _CHUNK

_mesh = plsc.VectorSubcoreMesh(core_axis_name="c", subcore_axis_name="s")


@functools.partial(
    pl.kernel,
    mesh=_mesh,
    out_type=jax.ShapeDtypeStruct((_TOTAL, _VOCAB), jnp.float32),
    scratch_types=[
        pltpu.VMEM((_NCHUNK, _CHUNK), jnp.int32),
        pltpu.VMEM((2, _CHUNK, _VOCAB), jnp.float32),
        pltpu.SemaphoreType.DMA,
        pltpu.SemaphoreType.DMA,
    ],
    compiler_params=pltpu.CompilerParams(use_tc_tiling_on_sc=False),
)
def _embed(idx_hbm, table_hbm, out_hbm, idx_v, rows_v, sem0, sem1):
    wid = lax.axis_index("s") * 2 + lax.axis_index("c")
    base = wid * _PER_W
    pltpu.sync_copy(idx_hbm.at[wid], idx_v)

    pltpu.async_copy(table_hbm.at[idx_v.at[0]], rows_v.at[0], sem0)

    def body(p, carry):
        g0 = p * 2
        pltpu.make_async_copy(table_hbm.at[idx_v.at[g0]], rows_v.at[0], sem0).wait()
        pltpu.async_copy(table_hbm.at[idx_v.at[g0 + 1]], rows_v.at[1], sem1)
        pltpu.sync_copy(rows_v.at[0], out_hbm.at[pl.ds(base + g0 * _CHUNK, _CHUNK)])

        pltpu.make_async_copy(table_hbm.at[idx_v.at[g0 + 1]], rows_v.at[1], sem1).wait()

        @pl.when(g0 + 2 < _NCHUNK)
        def _():
            pltpu.async_copy(table_hbm.at[idx_v.at[g0 + 2]], rows_v.at[0], sem0)

        pltpu.sync_copy(
            rows_v.at[1], out_hbm.at[pl.ds(base + (g0 + 1) * _CHUNK, _CHUNK)]
        )
        return carry

    lax.fori_loop(0, _NCHUNK // 2, body, 0)


def kernel(idx, targets, token_embedding_table):
    del targets
    idx3 = idx.reshape(_NW, _NCHUNK, _CHUNK).astype(jnp.int32)
    out = _embed(idx3, token_embedding_table)
    return out.reshape(_B, _L, _VOCAB)


# speedup vs baseline: 1.1415x; 1.1098x over previous
import functools

import jax
import jax.numpy as jnp
from jax import lax
from jax.experimental import pallas as pl
from jax.experimental.pallas import tpu as pltpu
from jax.experimental.pallas import tpu_sc as plsc

_VOCAB = 1000
_B = 1024
_L = 50
_TOTAL = _B * _L
_NW = 32
_PER_W = _TOTAL // _NW
_CHUNK = 32
_NCHUNK = _PER_W // _CHUNK
_SLAB = 64

_mesh = plsc.VectorSubcoreMesh(core_axis_name="c", subcore_axis_name="s")


@functools.partial(
    pl.kernel,
    mesh=_mesh,
    out_type=jax.ShapeDtypeStruct((_TOTAL, _VOCAB), jnp.float32),
    scratch_types=[
        pltpu.VMEM((_NCHUNK, _CHUNK), jnp.int32),
        pltpu.VMEM((2, _CHUNK, _VOCAB), jnp.float32),
        pltpu.VMEM_SHARED((_VOCAB, _VOCAB), jnp.float32),
        pltpu.SemaphoreType.DMA,
        pltpu.SemaphoreType.DMA,
    ],
    compiler_params=pltpu.CompilerParams(use_tc_tiling_on_sc=False),
)
def _embed(idx_hbm, table_hbm, out_hbm, idx_v, rows_v, table_sh, sem0, sem1):
    cid = lax.axis_index("c")
    sid = lax.axis_index("s")
    wid = sid * 2 + cid
    base = wid * _PER_W

    @pl.when(sid < 15)
    def _():
        pltpu.sync_copy(
            table_hbm.at[pl.ds(sid * _SLAB, _SLAB)],
            table_sh.at[pl.ds(sid * _SLAB, _SLAB)],
        )

    @pl.when(sid == 15)
    def _():
        pltpu.sync_copy(
            table_hbm.at[pl.ds(15 * _SLAB, _VOCAB - 15 * _SLAB)],
            table_sh.at[pl.ds(15 * _SLAB, _VOCAB - 15 * _SLAB)],
        )

    pltpu.sync_copy(idx_hbm.at[wid], idx_v)
    plsc.subcore_barrier()

    pltpu.async_copy(table_sh.at[idx_v.at[0]], rows_v.at[0], sem0)

    def body(p, carry):
        g0 = p * 2
        pltpu.make_async_copy(table_sh.at[idx_v.at[g0]], rows_v.at[0], sem0).wait()
        pltpu.async_copy(table_sh.at[idx_v.at[g0 + 1]], rows_v.at[1], sem1)
        pltpu.sync_copy(rows_v.at[0], out_hbm.at[pl.ds(base + g0 * _CHUNK, _CHUNK)])

        pltpu.make_async_copy(table_sh.at[idx_v.at[g0 + 1]], rows_v.at[1], sem1).wait()

        @pl.when(g0 + 2 < _NCHUNK)
        def _():
            pltpu.async_copy(table_sh.at[idx_v.at[g0 + 2]], rows_v.at[0], sem0)

        pltpu.sync_copy(
            rows_v.at[1], out_hbm.at[pl.ds(base + (g0 + 1) * _CHUNK, _CHUNK)]
        )
        return carry

    lax.fori_loop(0, _NCHUNK // 2, body, 0)


def kernel(idx, targets, token_embedding_table):
    del targets
    idx3 = idx.reshape(_NW, _NCHUNK, _CHUNK).astype(jnp.int32)
    out = _embed(idx3, token_embedding_table)
    return out.reshape(_B, _L, _VOCAB)
